# Initial kernel scaffold; baseline (speedup 1.0000x reference)
import jax
import jax.numpy as jnp
from jax.experimental import pallas as pl


def kernel(input, stack_idx, weight, bias):
    raise NotImplementedError("write your pallas kernel here")


# speedup vs baseline: 28.9024x; 28.9024x over previous
import functools

import jax
import jax.numpy as jnp
from jax import lax
from jax.experimental import pallas as pl
from jax.experimental.pallas import tpu as pltpu
from jax.experimental.pallas import tpu_sc as plsc

BT = 128
NB = 80
S = NB * BT
CH = 64


def _sc_gather(x, row_ids_3d, S, IN):
    info = plsc.get_sparse_core_info()
    NC, NS = info.num_cores, info.num_subcores
    NW = NC * NS
    b_per_w = S // NW
    n_chunks = b_per_w // CH
    mesh = plsc.VectorSubcoreMesh(core_axis_name="c", subcore_axis_name="s")

    @functools.partial(
        pl.kernel,
        mesh=mesh,
        out_type=jax.ShapeDtypeStruct((S, IN), jnp.float32),
        scratch_types=[
            pltpu.VMEM((CH,), jnp.int32),
            pltpu.VMEM((CH, IN), jnp.float32),
            pltpu.SemaphoreType.DMA,
        ],
    )
    def gather_k(x_hbm, idx_hbm, out_hbm, idx_v, rows_v, sem):
        wid = lax.axis_index("s") * NC + lax.axis_index("c")

        def chunk(k, carry):
            b0 = wid * b_per_w + k * CH
            pltpu.sync_copy(idx_hbm.at[wid, k], idx_v)
            pltpu.async_copy(x_hbm.at[idx_v], rows_v, sem).wait()
            pltpu.sync_copy(rows_v, out_hbm.at[pl.ds(b0, CH)])
            return carry

        lax.fori_loop(0, n_chunks, chunk, 0)

    return gather_k(x, row_ids_3d)


def _sc_scatter(y_sorted, row_ids_3d, B, OUT):
    info = plsc.get_sparse_core_info()
    NC, NS = info.num_cores, info.num_subcores
    NW = NC * NS
    S_rows = y_sorted.shape[0]
    b_per_w = S_rows // NW
    n_chunks = b_per_w // CH
    mesh = plsc.VectorSubcoreMesh(core_axis_name="c", subcore_axis_name="s")

    @functools.partial(
        pl.kernel,
        mesh=mesh,
        out_type=jax.ShapeDtypeStruct((B, OUT), jnp.float32),
        scratch_types=[
            pltpu.VMEM((CH,), jnp.int32),
            pltpu.VMEM((CH, OUT), jnp.float32),
            pltpu.SemaphoreType.DMA,
        ],
    )
    def scatter_k(ys_hbm, idx_hbm, out_hbm, idx_v, rows_v, sem):
        wid = lax.axis_index("s") * NC + lax.axis_index("c")

        def chunk(k, carry):
            b0 = wid * b_per_w + k * CH
            pltpu.sync_copy(idx_hbm.at[wid, k], idx_v)
            pltpu.sync_copy(ys_hbm.at[pl.ds(b0, CH)], rows_v)
            pltpu.async_copy(rows_v, out_hbm.at[idx_v], sem).wait()
            return carry

        lax.fori_loop(0, n_chunks, chunk, 0)

    return scatter_k(y_sorted, row_ids_3d)


def _mm_body(be_ref, x_ref, w_ref, b_ref, o_ref):
    acc = lax.dot_general(
        x_ref[...], w_ref[0],
        (((1,), (1,)), ((), ())),
        preferred_element_type=jnp.float32,
    )
    o_ref[...] = acc + b_ref[0]


def _tc_grouped_matmul(x_sorted, blk_e, weight, bias, OUT, IN):
    grid_spec = pltpu.PrefetchScalarGridSpec(
        num_scalar_prefetch=1,
        grid=(NB,),
        in_specs=[
            pl.BlockSpec((BT, IN), lambda i, be: (i, 0)),
            pl.BlockSpec((1, OUT, IN), lambda i, be: (be[i], 0, 0)),
            pl.BlockSpec((1, 1, OUT), lambda i, be: (be[i], 0, 0)),
        ],
        out_specs=pl.BlockSpec((BT, OUT), lambda i, be: (i, 0)),
    )
    return pl.pallas_call(
        _mm_body,
        grid_spec=grid_spec,
        out_shape=jax.ShapeDtypeStruct((S, OUT), jnp.float32),
        compiler_params=pltpu.CompilerParams(
            dimension_semantics=("arbitrary",),
        ),
    )(blk_e, x_sorted, weight, bias.reshape(bias.shape[0], 1, OUT))


def _routing(stack_idx, B, E):
    e = stack_idx.astype(jnp.int32)
    onehot = (e[:, None] == jnp.arange(E, dtype=jnp.int32)[None, :]).astype(jnp.int32)
    c = jnp.sum(onehot, axis=0)
    off = jnp.cumsum(c) - c
    rank = jnp.take_along_axis(jnp.cumsum(onehot, axis=0), e[:, None], axis=1)[:, 0] - 1
    pos_sorted = off[e] + rank
    order = jnp.zeros((B,), jnp.int32).at[pos_sorted].set(
        jnp.arange(B, dtype=jnp.int32))

    nb = (c + BT - 1) // BT
    fb_end = jnp.cumsum(nb)
    fb = fb_end - nb
    nb_used = fb_end[-1]

    blk = jnp.arange(NB, dtype=jnp.int32)
    blk_e = jnp.searchsorted(fb_end, blk, side="right").astype(jnp.int32)
    t0 = order[0]
    e0 = e[t0]
    used = blk < nb_used
    blk_e = jnp.where(used, jnp.minimum(blk_e, E - 1), e0)

    j = jnp.arange(BT, dtype=jnp.int32)
    pos = (blk - fb[blk_e])[:, None] * BT + j[None, :]
    pos_c = jnp.minimum(pos, c[blk_e][:, None] - 1)
    src_sorted = off[blk_e][:, None] + pos_c
    row_ids = jnp.where(used[:, None], order[src_sorted], t0)
    return blk_e, row_ids.reshape(-1).astype(jnp.int32)


def kernel(input, stack_idx, weight, bias):
    B, IN = input.shape
    E, OUT, _ = weight.shape
    blk_e, row_ids = _routing(stack_idx, B, E)

    info = plsc.get_sparse_core_info()
    NW = info.num_cores * info.num_subcores
    row_ids_3d = row_ids.reshape(NW, S // NW // CH, CH)

    x_sorted = _sc_gather(input, row_ids_3d, S, IN)
    y_sorted = _tc_grouped_matmul(x_sorted, blk_e, weight, bias, OUT, IN)
    return _sc_scatter(y_sorted, row_ids_3d, B, OUT)


# speedup vs baseline: 31.3921x; 1.0861x over previous
import functools

import jax
import jax.numpy as jnp
from jax import lax
from jax.experimental import pallas as pl
from jax.experimental.pallas import tpu as pltpu
from jax.experimental.pallas import tpu_sc as plsc

BT = 128
NB = 80
S = NB * BT
CH = 64


def _sc_gather(x, row_ids_3d, S, IN):
    info = plsc.get_sparse_core_info()
    NC, NS = info.num_cores, info.num_subcores
    NW = NC * NS
    b_per_w = S // NW
    n_chunks = b_per_w // CH
    mesh = plsc.VectorSubcoreMesh(core_axis_name="c", subcore_axis_name="s")

    @functools.partial(
        pl.kernel,
        mesh=mesh,
        out_type=jax.ShapeDtypeStruct((S, IN), jnp.float32),
        scratch_types=[
            pltpu.VMEM((CH,), jnp.int32),
            pltpu.VMEM((CH, IN), jnp.float32),
            pltpu.SemaphoreType.DMA,
        ],
    )
    def gather_k(x_hbm, idx_hbm, out_hbm, idx_v, rows_v, sem):
        wid = lax.axis_index("s") * NC + lax.axis_index("c")

        def chunk(k, carry):
            b0 = wid * b_per_w + k * CH
            pltpu.sync_copy(idx_hbm.at[wid, k], idx_v)
            pltpu.async_copy(x_hbm.at[idx_v], rows_v, sem).wait()
            pltpu.sync_copy(rows_v, out_hbm.at[pl.ds(b0, CH)])
            return carry

        lax.fori_loop(0, n_chunks, chunk, 0)

    return gather_k(x, row_ids_3d)


def _sc_scatter(y_sorted, row_ids_3d, B, OUT):
    info = plsc.get_sparse_core_info()
    NC, NS = info.num_cores, info.num_subcores
    NW = NC * NS
    S_rows = y_sorted.shape[0]
    b_per_w = S_rows // NW
    n_chunks = b_per_w // CH
    mesh = plsc.VectorSubcoreMesh(core_axis_name="c", subcore_axis_name="s")

    @functools.partial(
        pl.kernel,
        mesh=mesh,
        out_type=jax.ShapeDtypeStruct((B, OUT), jnp.float32),
        scratch_types=[
            pltpu.VMEM((CH,), jnp.int32),
            pltpu.VMEM((CH, OUT), jnp.float32),
            pltpu.SemaphoreType.DMA,
        ],
    )
    def scatter_k(ys_hbm, idx_hbm, out_hbm, idx_v, rows_v, sem):
        wid = lax.axis_index("s") * NC + lax.axis_index("c")

        def chunk(k, carry):
            b0 = wid * b_per_w + k * CH
            pltpu.sync_copy(idx_hbm.at[wid, k], idx_v)
            pltpu.sync_copy(ys_hbm.at[pl.ds(b0, CH)], rows_v)
            pltpu.async_copy(rows_v, out_hbm.at[idx_v], sem).wait()
            return carry

        lax.fori_loop(0, n_chunks, chunk, 0)

    return scatter_k(y_sorted, row_ids_3d)


def _mm_body(be_ref, x_ref, w_ref, b_ref, o_ref):
    acc = lax.dot_general(
        x_ref[...], w_ref[0],
        (((1,), (1,)), ((), ())),
        preferred_element_type=jnp.float32,
    )
    o_ref[...] = acc + b_ref[0]


def _tc_grouped_matmul(x_sorted, blk_e, weight, bias, OUT, IN):
    grid_spec = pltpu.PrefetchScalarGridSpec(
        num_scalar_prefetch=1,
        grid=(NB,),
        in_specs=[
            pl.BlockSpec((BT, IN), lambda i, be: (i, 0)),
            pl.BlockSpec((1, OUT, IN), lambda i, be: (be[i], 0, 0)),
            pl.BlockSpec((1, 1, OUT), lambda i, be: (be[i], 0, 0)),
        ],
        out_specs=pl.BlockSpec((BT, OUT), lambda i, be: (i, 0)),
    )
    return pl.pallas_call(
        _mm_body,
        grid_spec=grid_spec,
        out_shape=jax.ShapeDtypeStruct((S, OUT), jnp.float32),
        compiler_params=pltpu.CompilerParams(
            dimension_semantics=("arbitrary",),
        ),
    )(blk_e, x_sorted, weight, bias.reshape(bias.shape[0], 1, OUT))


def _routing(stack_idx, B, E):
    e = stack_idx.astype(jnp.int32)
    order = jnp.argsort(e).astype(jnp.int32)
    sorted_e = jnp.take(e, order)
    ar_e = jnp.arange(E, dtype=jnp.int32)
    off = jnp.searchsorted(sorted_e, ar_e, side="left").astype(jnp.int32)
    end = jnp.searchsorted(sorted_e, ar_e, side="right").astype(jnp.int32)
    c = end - off

    nb = (c + BT - 1) // BT
    fb_end = jnp.cumsum(nb)
    fb = fb_end - nb
    nb_used = fb_end[-1]

    blk = jnp.arange(NB, dtype=jnp.int32)
    blk_e = jnp.searchsorted(fb_end, blk, side="right").astype(jnp.int32)
    t0 = order[0]
    e0 = e[t0]
    used = blk < nb_used
    blk_e = jnp.where(used, jnp.minimum(blk_e, E - 1), e0)

    j = jnp.arange(BT, dtype=jnp.int32)
    pos = (blk - fb[blk_e])[:, None] * BT + j[None, :]
    pos_c = jnp.minimum(pos, c[blk_e][:, None] - 1)
    src_sorted = off[blk_e][:, None] + pos_c
    row_ids = jnp.where(used[:, None], order[src_sorted], t0)
    return blk_e, row_ids.reshape(-1).astype(jnp.int32)


def kernel(input, stack_idx, weight, bias):
    B, IN = input.shape
    E, OUT, _ = weight.shape
    blk_e, row_ids = _routing(stack_idx, B, E)

    info = plsc.get_sparse_core_info()
    NW = info.num_cores * info.num_subcores
    row_ids_3d = row_ids.reshape(NW, S // NW // CH, CH)

    x_sorted = _sc_gather(input, row_ids_3d, S, IN)
    y_sorted = _tc_grouped_matmul(x_sorted, blk_e, weight, bias, OUT, IN)
    return _sc_scatter(y_sorted, row_ids_3d, B, OUT)


# speedup vs baseline: 43.8708x; 1.3975x over previous
import functools

import jax
import jax.numpy as jnp
from jax import lax
from jax.experimental import pallas as pl
from jax.experimental.pallas import tpu as pltpu
from jax.experimental.pallas import tpu_sc as plsc

BT = 128
NB = 80
S = NB * BT
CH = 64


def _sc_scatter_x(x, slots_3d, S, IN):
    info = plsc.get_sparse_core_info()
    NC, NS = info.num_cores, info.num_subcores
    NW = NC * NS
    B = x.shape[0]
    b_per_w = B // NW
    n_chunks = b_per_w // CH
    mesh = plsc.VectorSubcoreMesh(core_axis_name="c", subcore_axis_name="s")

    @functools.partial(
        pl.kernel,
        mesh=mesh,
        out_type=jax.ShapeDtypeStruct((S, IN), jnp.float32),
        scratch_types=[
            pltpu.VMEM((CH,), jnp.int32),
            pltpu.VMEM((CH, IN), jnp.float32),
            pltpu.SemaphoreType.DMA,
        ],
    )
    def scatter_k(x_hbm, idx_hbm, out_hbm, idx_v, rows_v, sem):
        wid = lax.axis_index("s") * NC + lax.axis_index("c")

        def chunk(k, carry):
            b0 = wid * b_per_w + k * CH
            pltpu.sync_copy(idx_hbm.at[wid, k], idx_v)
            pltpu.sync_copy(x_hbm.at[pl.ds(b0, CH)], rows_v)
            pltpu.async_copy(rows_v, out_hbm.at[idx_v], sem).wait()
            return carry

        lax.fori_loop(0, n_chunks, chunk, 0)

    return scatter_k(x, slots_3d)


def _sc_gather_y(y_sorted, slots_3d, B, OUT):
    info = plsc.get_sparse_core_info()
    NC, NS = info.num_cores, info.num_subcores
    NW = NC * NS
    b_per_w = B // NW
    n_chunks = b_per_w // CH
    mesh = plsc.VectorSubcoreMesh(core_axis_name="c", subcore_axis_name="s")

    @functools.partial(
        pl.kernel,
        mesh=mesh,
        out_type=jax.ShapeDtypeStruct((B, OUT), jnp.float32),
        scratch_types=[
            pltpu.VMEM((CH,), jnp.int32),
            pltpu.VMEM((CH, OUT), jnp.float32),
            pltpu.SemaphoreType.DMA,
        ],
    )
    def gather_k(ys_hbm, idx_hbm, out_hbm, idx_v, rows_v, sem):
        wid = lax.axis_index("s") * NC + lax.axis_index("c")

        def chunk(k, carry):
            b0 = wid * b_per_w + k * CH
            pltpu.sync_copy(idx_hbm.at[wid, k], idx_v)
            pltpu.async_copy(ys_hbm.at[idx_v], rows_v, sem).wait()
            pltpu.sync_copy(rows_v, out_hbm.at[pl.ds(b0, CH)])
            return carry

        lax.fori_loop(0, n_chunks, chunk, 0)

    return gather_k(y_sorted, slots_3d)


def _mm_body(be_ref, x_ref, w_ref, b_ref, o_ref):
    acc = lax.dot_general(
        x_ref[...], w_ref[0],
        (((1,), (1,)), ((), ())),
        preferred_element_type=jnp.float32,
    )
    o_ref[...] = acc + b_ref[0]


def _tc_grouped_matmul(x_sorted, blk_e, weight, bias, OUT, IN):
    grid_spec = pltpu.PrefetchScalarGridSpec(
        num_scalar_prefetch=1,
        grid=(NB,),
        in_specs=[
            pl.BlockSpec((BT, IN), lambda i, be: (i, 0)),
            pl.BlockSpec((1, OUT, IN), lambda i, be: (be[i], 0, 0)),
            pl.BlockSpec((1, 1, OUT), lambda i, be: (be[i], 0, 0)),
        ],
        out_specs=pl.BlockSpec((BT, OUT), lambda i, be: (i, 0)),
    )
    return pl.pallas_call(
        _mm_body,
        grid_spec=grid_spec,
        out_shape=jax.ShapeDtypeStruct((S, OUT), jnp.float32),
        compiler_params=pltpu.CompilerParams(
            dimension_semantics=("arbitrary",),
        ),
    )(blk_e, x_sorted, weight, bias.reshape(bias.shape[0], 1, OUT))


def _routing(stack_idx, B, E):
    NCH = 64
    CL = B // NCH
    e2 = stack_idx.astype(jnp.int32).reshape(NCH, CL)
    ar_e = jnp.arange(E, dtype=jnp.int32)
    oh = (e2[:, :, None] == ar_e[None, None, :]).astype(jnp.int32)
    within = jnp.cumsum(oh, axis=1)
    chunk_hist = within[:, -1, :]
    prefix = jnp.cumsum(chunk_hist, axis=0) - chunk_hist
    c = jnp.sum(chunk_hist, axis=0)

    nb = (c + BT - 1) // BT
    fb_end = jnp.cumsum(nb)
    fb = fb_end - nb
    blk = jnp.arange(NB, dtype=jnp.int32)
    blk_e = jnp.minimum(
        jnp.searchsorted(fb_end, blk, side="right").astype(jnp.int32), E - 1)

    base = fb[None, :] * BT + prefix
    rank_in = jnp.take_along_axis(within, e2[:, :, None], axis=2)[:, :, 0] - 1
    base_tok = jnp.take_along_axis(base, e2, axis=1)
    slots = (base_tok + rank_in).reshape(B).astype(jnp.int32)
    return blk_e, slots


def kernel(input, stack_idx, weight, bias):
    B, IN = input.shape
    E, OUT, _ = weight.shape
    blk_e, slots = _routing(stack_idx, B, E)

    info = plsc.get_sparse_core_info()
    NW = info.num_cores * info.num_subcores
    slots_3d = slots.reshape(NW, B // NW // CH, CH)

    x_sorted = _sc_scatter_x(input, slots_3d, S, IN)
    y_sorted = _tc_grouped_matmul(x_sorted, blk_e, weight, bias, OUT, IN)
    return _sc_gather_y(y_sorted, slots_3d, B, OUT)


# speedup vs baseline: 54.3872x; 1.2397x over previous
import functools

import jax
import jax.numpy as jnp
from jax import lax
from jax.experimental import pallas as pl
from jax.experimental.pallas import tpu as pltpu
from jax.experimental.pallas import tpu_sc as plsc

BT = 128
NB = 80
S = NB * BT
CH = 64


def _sc_scatter_x(x, slots_3d, S, IN):
    info = plsc.get_sparse_core_info()
    NC, NS = info.num_cores, info.num_subcores
    NW = NC * NS
    B = x.shape[0]
    b_per_w = B // NW
    n_chunks = b_per_w // CH
    mesh = plsc.VectorSubcoreMesh(core_axis_name="c", subcore_axis_name="s")

    @functools.partial(
        pl.kernel,
        mesh=mesh,
        out_type=jax.ShapeDtypeStruct((S, IN), jnp.float32),
        scratch_types=[
            pltpu.VMEM((CH,), jnp.int32),
            pltpu.VMEM((CH, IN), jnp.float32),
            pltpu.SemaphoreType.DMA,
        ],
    )
    def scatter_k(x_hbm, idx_hbm, out_hbm, idx_v, rows_v, sem):
        wid = lax.axis_index("s") * NC + lax.axis_index("c")

        def chunk(k, carry):
            b0 = wid * b_per_w + k * CH
            pltpu.sync_copy(idx_hbm.at[wid, k], idx_v)
            pltpu.sync_copy(x_hbm.at[pl.ds(b0, CH)], rows_v)
            pltpu.async_copy(rows_v, out_hbm.at[idx_v], sem).wait()
            return carry

        lax.fori_loop(0, n_chunks, chunk, 0)

    return scatter_k(x, slots_3d)


def _sc_gather_y(y_sorted, slots_3d, B, OUT):
    info = plsc.get_sparse_core_info()
    NC, NS = info.num_cores, info.num_subcores
    NW = NC * NS
    b_per_w = B // NW
    n_chunks = b_per_w // CH
    mesh = plsc.VectorSubcoreMesh(core_axis_name="c", subcore_axis_name="s")

    @functools.partial(
        pl.kernel,
        mesh=mesh,
        out_type=jax.ShapeDtypeStruct((B, OUT), jnp.float32),
        scratch_types=[
            pltpu.VMEM((CH,), jnp.int32),
            pltpu.VMEM((CH, OUT), jnp.float32),
            pltpu.SemaphoreType.DMA,
        ],
    )
    def gather_k(ys_hbm, idx_hbm, out_hbm, idx_v, rows_v, sem):
        wid = lax.axis_index("s") * NC + lax.axis_index("c")

        def chunk(k, carry):
            b0 = wid * b_per_w + k * CH
            pltpu.sync_copy(idx_hbm.at[wid, k], idx_v)
            pltpu.async_copy(ys_hbm.at[idx_v], rows_v, sem).wait()
            pltpu.sync_copy(rows_v, out_hbm.at[pl.ds(b0, CH)])
            return carry

        lax.fori_loop(0, n_chunks, chunk, 0)

    return gather_k(y_sorted, slots_3d)


def _mm_body(be_ref, x_ref, w_ref, b_ref, o_ref):
    acc = lax.dot_general(
        x_ref[...], w_ref[0],
        (((1,), (1,)), ((), ())),
        preferred_element_type=jnp.float32,
    )
    o_ref[...] = acc + b_ref[0]


def _tc_grouped_matmul(x_sorted, blk_e, weight, bias, OUT, IN):
    grid_spec = pltpu.PrefetchScalarGridSpec(
        num_scalar_prefetch=1,
        grid=(NB,),
        in_specs=[
            pl.BlockSpec((BT, IN), lambda i, be: (i, 0)),
            pl.BlockSpec((1, OUT, IN), lambda i, be: (be[i], 0, 0)),
            pl.BlockSpec((1, 1, OUT), lambda i, be: (be[i], 0, 0)),
        ],
        out_specs=pl.BlockSpec((BT, OUT), lambda i, be: (i, 0)),
    )
    return pl.pallas_call(
        _mm_body,
        grid_spec=grid_spec,
        out_shape=jax.ShapeDtypeStruct((S, OUT), jnp.float32),
        compiler_params=pltpu.CompilerParams(
            dimension_semantics=("arbitrary",),
        ),
    )(blk_e, x_sorted, weight, bias.reshape(bias.shape[0], 1, OUT))


def _routing(stack_idx, B, E):
    NCH = 64
    CL = B // NCH
    e2 = stack_idx.astype(jnp.int32).reshape(NCH, CL)
    ar_e = jnp.arange(E, dtype=jnp.int32)
    oh = (e2[:, None, :] == ar_e[None, :, None]).astype(jnp.int32)
    within = jnp.cumsum(oh, axis=2)
    chunk_hist = within[:, :, -1]
    prefix = jnp.cumsum(chunk_hist, axis=0) - chunk_hist
    c = jnp.sum(chunk_hist, axis=0)

    nb = (c + BT - 1) // BT
    fb_end = jnp.cumsum(nb)
    fb = fb_end - nb
    blk = jnp.arange(NB, dtype=jnp.int32)
    blk_e = jnp.minimum(
        jnp.searchsorted(fb_end, blk, side="right").astype(jnp.int32), E - 1)

    base = fb[None, :] * BT + prefix
    slots = jnp.sum((base[:, :, None] + within - 1) * oh, axis=1)
    return blk_e, slots.reshape(B).astype(jnp.int32)


def kernel(input, stack_idx, weight, bias):
    B, IN = input.shape
    E, OUT, _ = weight.shape
    blk_e, slots = _routing(stack_idx, B, E)

    info = plsc.get_sparse_core_info()
    NW = info.num_cores * info.num_subcores
    slots_3d = slots.reshape(NW, B // NW // CH, CH)

    x_sorted = _sc_scatter_x(input, slots_3d, S, IN)
    y_sorted = _tc_grouped_matmul(x_sorted, blk_e, weight, bias, OUT, IN)
    return _sc_gather_y(y_sorted, slots_3d, B, OUT)


# speedup vs baseline: 58.6913x; 1.0791x over previous
import functools

import jax
import jax.numpy as jnp
from jax import lax
from jax.experimental import pallas as pl
from jax.experimental.pallas import tpu as pltpu
from jax.experimental.pallas import tpu_sc as plsc

BT = 128
NB = 80
S = NB * BT
CH = 64


def _sc_scatter_x(x, slots_3d, S, IN):
    info = plsc.get_sparse_core_info()
    NC, NS = info.num_cores, info.num_subcores
    NW = NC * NS
    B = x.shape[0]
    b_per_w = B // NW
    n_chunks = b_per_w // CH
    mesh = plsc.VectorSubcoreMesh(core_axis_name="c", subcore_axis_name="s")

    @functools.partial(
        pl.kernel,
        mesh=mesh,
        out_type=jax.ShapeDtypeStruct((S, IN), jnp.float32),
        scratch_types=[
            pltpu.VMEM((CH,), jnp.int32),
            pltpu.VMEM((CH, IN), jnp.float32),
            pltpu.SemaphoreType.DMA,
        ],
    )
    def scatter_k(x_hbm, idx_hbm, out_hbm, idx_v, rows_v, sem):
        wid = lax.axis_index("s") * NC + lax.axis_index("c")

        def chunk(k, carry):
            b0 = wid * b_per_w + k * CH
            pltpu.sync_copy(idx_hbm.at[wid, k], idx_v)
            pltpu.sync_copy(x_hbm.at[pl.ds(b0, CH)], rows_v)
            pltpu.async_copy(rows_v, out_hbm.at[idx_v], sem).wait()
            return carry

        lax.fori_loop(0, n_chunks, chunk, 0)

    return scatter_k(x, slots_3d)


def _sc_gather_y(y_sorted, slots_3d, B, OUT):
    info = plsc.get_sparse_core_info()
    NC, NS = info.num_cores, info.num_subcores
    NW = NC * NS
    b_per_w = B // NW
    n_chunks = b_per_w // CH
    mesh = plsc.VectorSubcoreMesh(core_axis_name="c", subcore_axis_name="s")

    @functools.partial(
        pl.kernel,
        mesh=mesh,
        out_type=jax.ShapeDtypeStruct((B, OUT), jnp.float32),
        scratch_types=[
            pltpu.VMEM((CH,), jnp.int32),
            pltpu.VMEM((CH, OUT), jnp.float32),
            pltpu.SemaphoreType.DMA,
        ],
    )
    def gather_k(ys_hbm, idx_hbm, out_hbm, idx_v, rows_v, sem):
        wid = lax.axis_index("s") * NC + lax.axis_index("c")

        def chunk(k, carry):
            b0 = wid * b_per_w + k * CH
            pltpu.sync_copy(idx_hbm.at[wid, k], idx_v)
            pltpu.async_copy(ys_hbm.at[idx_v], rows_v, sem).wait()
            pltpu.sync_copy(rows_v, out_hbm.at[pl.ds(b0, CH)])
            return carry

        lax.fori_loop(0, n_chunks, chunk, 0)

    return gather_k(y_sorted, slots_3d)


def _mm_body(be_ref, x_ref, w_ref, b_ref, o_ref):
    acc = lax.dot_general(
        x_ref[...], w_ref[0],
        (((1,), (1,)), ((), ())),
        preferred_element_type=jnp.float32,
    )
    o_ref[...] = acc + b_ref[0]


def _tc_grouped_matmul(x_sorted, blk_e, weight, bias, OUT, IN):
    grid_spec = pltpu.PrefetchScalarGridSpec(
        num_scalar_prefetch=1,
        grid=(NB,),
        in_specs=[
            pl.BlockSpec((BT, IN), lambda i, be: (i, 0)),
            pl.BlockSpec((1, OUT, IN), lambda i, be: (be[i], 0, 0)),
            pl.BlockSpec((1, 1, OUT), lambda i, be: (be[i], 0, 0)),
        ],
        out_specs=pl.BlockSpec((BT, OUT), lambda i, be: (i, 0)),
    )
    return pl.pallas_call(
        _mm_body,
        grid_spec=grid_spec,
        out_shape=jax.ShapeDtypeStruct((S, OUT), jnp.float32),
        compiler_params=pltpu.CompilerParams(
            dimension_semantics=("arbitrary",),
        ),
    )(blk_e, x_sorted, weight, bias.reshape(bias.shape[0], 1, OUT))


def _routing(stack_idx, B, E):
    NCH = 64
    CL = B // NCH
    e2 = stack_idx.astype(jnp.int32).reshape(NCH, CL)
    ar_e = jnp.arange(E, dtype=jnp.int32)
    oh = (e2[:, None, :] == ar_e[None, :, None]).astype(jnp.float32)
    tri = (jnp.arange(CL)[:, None] <= jnp.arange(CL)[None, :]).astype(jnp.float32)
    within = lax.dot_general(
        oh.reshape(NCH * E, CL), tri, (((1,), (0,)), ((), ())),
        preferred_element_type=jnp.float32).reshape(NCH, E, CL)
    chunk_hist = within[:, :, -1].astype(jnp.int32)
    prefix = jnp.cumsum(chunk_hist, axis=0) - chunk_hist
    c = jnp.sum(chunk_hist, axis=0)

    nb = (c + BT - 1) // BT
    fb_end = jnp.cumsum(nb)
    fb = fb_end - nb
    blk = jnp.arange(NB, dtype=jnp.int32)
    blk_e = jnp.minimum(
        jnp.searchsorted(fb_end, blk, side="right").astype(jnp.int32), E - 1)

    base = (fb[None, :] * BT + prefix).astype(jnp.float32)
    slots = jnp.sum((base[:, :, None] + within - 1.0) * oh, axis=1)
    return blk_e, slots.reshape(B).astype(jnp.int32)


def kernel(input, stack_idx, weight, bias):
    B, IN = input.shape
    E, OUT, _ = weight.shape
    blk_e, slots = _routing(stack_idx, B, E)

    info = plsc.get_sparse_core_info()
    NW = info.num_cores * info.num_subcores
    slots_3d = slots.reshape(NW, B // NW // CH, CH)

    x_sorted = _sc_scatter_x(input, slots_3d, S, IN)
    y_sorted = _tc_grouped_matmul(x_sorted, blk_e, weight, bias, OUT, IN)
    return _sc_gather_y(y_sorted, slots_3d, B, OUT)


# speedup vs baseline: 68.8856x; 1.1737x over previous
import functools

import jax
import jax.numpy as jnp
from jax import lax
from jax.experimental import pallas as pl
from jax.experimental.pallas import tpu as pltpu
from jax.experimental.pallas import tpu_sc as plsc

BT = 128
NB = 80
S = NB * BT
CH = 64


def _sc_scatter_x(x, slots_3d, S, IN):
    info = plsc.get_sparse_core_info()
    NC, NS = info.num_cores, info.num_subcores
    NW = NC * NS
    B = x.shape[0]
    b_per_w = B // NW
    n_chunks = b_per_w // CH
    mesh = plsc.VectorSubcoreMesh(core_axis_name="c", subcore_axis_name="s")

    @functools.partial(
        pl.kernel,
        mesh=mesh,
        out_type=jax.ShapeDtypeStruct((S, IN), jnp.float32),
        scratch_types=[
            pltpu.VMEM((CH,), jnp.int32),
            pltpu.VMEM((CH, IN), jnp.float32),
            pltpu.SemaphoreType.DMA,
        ],
    )
    def scatter_k(x_hbm, idx_hbm, out_hbm, idx_v, rows_v, sem):
        wid = lax.axis_index("s") * NC + lax.axis_index("c")

        def chunk(k, carry):
            b0 = wid * b_per_w + k * CH
            pltpu.sync_copy(idx_hbm.at[wid, k], idx_v)
            pltpu.sync_copy(x_hbm.at[pl.ds(b0, CH)], rows_v)
            pltpu.async_copy(rows_v, out_hbm.at[idx_v], sem).wait()
            return carry

        lax.fori_loop(0, n_chunks, chunk, 0)

    return scatter_k(x, slots_3d)


def _sc_gather_y(y_sorted, slots_3d, B, OUT):
    info = plsc.get_sparse_core_info()
    NC, NS = info.num_cores, info.num_subcores
    NW = NC * NS
    b_per_w = B // NW
    n_chunks = b_per_w // CH
    mesh = plsc.VectorSubcoreMesh(core_axis_name="c", subcore_axis_name="s")

    @functools.partial(
        pl.kernel,
        mesh=mesh,
        out_type=jax.ShapeDtypeStruct((B, OUT), jnp.float32),
        scratch_types=[
            pltpu.VMEM((CH,), jnp.int32),
            pltpu.VMEM((CH, OUT), jnp.float32),
            pltpu.SemaphoreType.DMA,
        ],
    )
    def gather_k(ys_hbm, idx_hbm, out_hbm, idx_v, rows_v, sem):
        wid = lax.axis_index("s") * NC + lax.axis_index("c")

        def chunk(k, carry):
            b0 = wid * b_per_w + k * CH
            pltpu.sync_copy(idx_hbm.at[wid, k], idx_v)
            pltpu.async_copy(ys_hbm.at[idx_v], rows_v, sem).wait()
            pltpu.sync_copy(rows_v, out_hbm.at[pl.ds(b0, CH)])
            return carry

        lax.fori_loop(0, n_chunks, chunk, 0)

    return gather_k(y_sorted, slots_3d)


def _mm_body(be_ref, x_ref, w_ref, b_ref, o_ref):
    acc = lax.dot_general(
        x_ref[...], w_ref[0],
        (((1,), (1,)), ((), ())),
        preferred_element_type=jnp.float32,
    )
    o_ref[...] = acc + b_ref[0]


def _tc_grouped_matmul(x_sorted, blk_e, weight, bias, OUT, IN):
    grid_spec = pltpu.PrefetchScalarGridSpec(
        num_scalar_prefetch=1,
        grid=(NB,),
        in_specs=[
            pl.BlockSpec((BT, IN), lambda i, be: (i, 0)),
            pl.BlockSpec((1, OUT, IN), lambda i, be: (be[i], 0, 0)),
            pl.BlockSpec((1, 1, OUT), lambda i, be: (be[i], 0, 0)),
        ],
        out_specs=pl.BlockSpec((BT, OUT), lambda i, be: (i, 0)),
    )
    return pl.pallas_call(
        _mm_body,
        grid_spec=grid_spec,
        out_shape=jax.ShapeDtypeStruct((S, OUT), jnp.float32),
        compiler_params=pltpu.CompilerParams(
            dimension_semantics=("arbitrary",),
        ),
    )(blk_e, x_sorted, weight, bias.reshape(bias.shape[0], 1, OUT))


def _routing(stack_idx, B, E):
    NCH = 64
    CL = B // NCH
    e2 = stack_idx.astype(jnp.int32).reshape(NCH, CL)
    ar_e = jnp.arange(E, dtype=jnp.int32)
    oh = (e2[:, None, :] == ar_e[None, :, None]).astype(jnp.float32)
    tri = (jnp.arange(CL)[:, None] <= jnp.arange(CL)[None, :]).astype(jnp.float32)
    within = lax.dot_general(
        oh.reshape(NCH * E, CL), tri, (((1,), (0,)), ((), ())),
        preferred_element_type=jnp.float32).reshape(NCH, E, CL)
    chunk_hist = within[:, :, -1].astype(jnp.int32)
    prefix = jnp.cumsum(chunk_hist, axis=0) - chunk_hist
    c = jnp.sum(chunk_hist, axis=0)

    nb = (c + BT - 1) // BT
    fb_end = jnp.cumsum(nb)
    fb = fb_end - nb
    blk = jnp.arange(NB, dtype=jnp.int32)
    blk_e = jnp.minimum(
        jnp.sum((blk[:, None] >= fb_end[None, :]).astype(jnp.int32), axis=1),
        E - 1)

    base = (fb[None, :] * BT + prefix).astype(jnp.float32)
    slots = jnp.sum((base[:, :, None] + within - 1.0) * oh, axis=1)
    return blk_e, slots.reshape(B).astype(jnp.int32)


def kernel(input, stack_idx, weight, bias):
    B, IN = input.shape
    E, OUT, _ = weight.shape
    blk_e, slots = _routing(stack_idx, B, E)

    info = plsc.get_sparse_core_info()
    NW = info.num_cores * info.num_subcores
    slots_3d = slots.reshape(NW, B // NW // CH, CH)

    x_sorted = _sc_scatter_x(input, slots_3d, S, IN)
    y_sorted = _tc_grouped_matmul(x_sorted, blk_e, weight, bias, OUT, IN)
    return _sc_gather_y(y_sorted, slots_3d, B, OUT)
